# baseline (speedup 1.0000x reference)
import functools

import jax
import jax.numpy as jnp
from jax import lax
from jax.experimental import pallas as pl
from jax.experimental.pallas import tpu as pltpu
from jax.experimental.pallas import tpu_sc as plsc

N = 4096
E = 65536
D = 128
C = 16
NN = N * N

NC = 2
NS = 16
NW = NC * NS
L = 16

CH = E // NW
CHR = CH // 128
SUB = 128
NSUB = CH // SUB
SUBR = SUB // 128

ROWBLK = 256
NBLK = N // ROWBLK

_mesh = plsc.VectorSubcoreMesh(core_axis_name="c", subcore_axis_name="s")
_sc_params = pltpu.CompilerParams(needs_layout_passes=False)


def _wid():
    cid = lax.axis_index("c")
    sid = lax.axis_index("s")
    return cid, sid, cid * NS + sid


def _vloop(n16, body):
    def step(i, carry):
        body(i)
        return carry
    lax.fori_loop(0, n16, step, 0)


def _k0_body(c0_ref, embed_ref, x_ref, wm_ref, w1_ref, p_ref, q_ref, z_ref):
    emb = embed_ref[...]
    p_ref[...] = lax.dot_general(
        emb, wm_ref[0:D, :], (((1,), (0,)), ((), ())),
        preferred_element_type=jnp.float32)
    q_ref[...] = lax.dot_general(
        emb, wm_ref[D:2 * D, :], (((1,), (0,)), ((), ())),
        preferred_element_type=jnp.float32) + c0_ref[0, 0]
    z_ref[...] = lax.dot_general(
        x_ref[...], w1_ref[...], (((1,), (0,)), ((), ())),
        preferred_element_type=jnp.float32)


def _k0(c0, embed, x, W_mask, W1):
    return pl.pallas_call(
        _k0_body,
        grid=(NBLK,),
        in_specs=[
            pl.BlockSpec(memory_space=pltpu.SMEM),
            pl.BlockSpec((ROWBLK, D), lambda i: (i, 0)),
            pl.BlockSpec((ROWBLK, D), lambda i: (i, 0)),
            pl.BlockSpec((3 * D, 1), lambda i: (0, 0)),
            pl.BlockSpec((D, D), lambda i: (0, 0)),
        ],
        out_specs=[
            pl.BlockSpec((ROWBLK, 1), lambda i: (i, 0)),
            pl.BlockSpec((ROWBLK, 1), lambda i: (i, 0)),
            pl.BlockSpec((ROWBLK, D), lambda i: (i, 0)),
        ],
        out_shape=[
            jax.ShapeDtypeStruct((N, 1), jnp.float32),
            jax.ShapeDtypeStruct((N, 1), jnp.float32),
            jax.ShapeDtypeStruct((N, D), jnp.float32),
        ],
    )(c0, embed, x, W_mask, W1)


BHASH = 1 << 20
BSLICE = BHASH // NS


def _hash(uk):
    return jnp.bitwise_and(jnp.bitwise_xor(uk, uk >> 11), BHASH - 1)


def _k1a_body(r_hbm, c_hbm, bloom_hbm,
              r_v, c_v, h_v, ones_v, zero_v, bloom_sh, sem):
    cid, sid, wid = _wid()
    base = wid * CHR

    def zbody(i):
        zero_v[pl.ds(i * L, L)] = jnp.zeros((L,), jnp.int32)
    _vloop(4096 // L, zbody)
    for j in range(BSLICE // 4096):
        pltpu.sync_copy(zero_v,
                        bloom_sh.at[pl.ds(sid * BSLICE + j * 4096, 4096)])

    def obody(t):
        i = t // 8
        j = lax.rem(t, 8)
        ones_v[i, pl.ds(j * L, L)] = jnp.zeros((L,), jnp.int32) + 1
    _vloop(CH // L, obody)

    pltpu.sync_copy(r_hbm.at[pl.ds(base, CHR), :], r_v)
    pltpu.sync_copy(c_hbm.at[pl.ds(base, CHR), :], c_v)

    def body(t):
        i = t // 8
        j = lax.rem(t, 8)
        rr = r_v[i, pl.ds(j * L, L)]
        cc = c_v[i, pl.ds(j * L, L)]
        uk = jnp.minimum(rr, cc) * N + jnp.maximum(rr, cc)
        h_v[i, pl.ds(j * L, L)] = _hash(uk)
    _vloop(CH // L, body)

    plsc.subcore_barrier()
    for j in range(CHR):
        pltpu.sync_copy(ones_v.at[j], bloom_sh.at[h_v.at[j]], add=True)
    plsc.subcore_barrier()
    pltpu.sync_copy(bloom_sh.at[pl.ds(sid * BSLICE, BSLICE)],
                    bloom_hbm.at[pl.ds(cid * BHASH + sid * BSLICE, BSLICE)])


def _k1a(r2, c2):
    return pl.kernel(
        _k1a_body,
        out_type=jax.ShapeDtypeStruct((NC * BHASH,), jnp.int32),
        mesh=_mesh,
        compiler_params=_sc_params,
        scratch_types=[
            pltpu.VMEM((CHR, 128), jnp.int32),
            pltpu.VMEM((CHR, 128), jnp.int32),
            pltpu.VMEM((CHR, 128), jnp.int32),
            pltpu.VMEM((CHR, 128), jnp.int32),
            pltpu.VMEM((4096,), jnp.int32),
            pltpu.VMEM_SHARED((BHASH,), jnp.int32),
            pltpu.SemaphoreType.DMA,
        ],
    )(r2, c2)


def _k1b_body(r_hbm, c_hbm, eids_hbm, bloom_hbm,
              darr_hbm, amb_hbm,
              r_v, c_v, eid_v, ukey_v, h_v, h2_v, b0_v, b1_v, amb_v,
              cukey_f, ceid_f, cukey_v, ceid_v, sem):
    cid, sid, wid = _wid()
    base = wid * CHR

    pltpu.sync_copy(r_hbm.at[pl.ds(base, CHR), :], r_v)
    pltpu.sync_copy(c_hbm.at[pl.ds(base, CHR), :], c_v)
    pltpu.sync_copy(eids_hbm.at[pl.ds(base, CHR), :], eid_v)

    def body(t):
        i = t // 8
        j = lax.rem(t, 8)
        rr = r_v[i, pl.ds(j * L, L)]
        cc = c_v[i, pl.ds(j * L, L)]
        uk = jnp.minimum(rr, cc) * N + jnp.maximum(rr, cc)
        h = _hash(uk)
        ukey_v[i, pl.ds(j * L, L)] = uk
        h_v[i, pl.ds(j * L, L)] = h
        h2_v[i, pl.ds(j * L, L)] = h + BHASH
    _vloop(CH // L, body)

    descs = ([pltpu.async_copy(bloom_hbm.at[h_v.at[j]], b0_v.at[j], sem)
              for j in range(CHR)]
             + [pltpu.async_copy(bloom_hbm.at[h2_v.at[j]], b1_v.at[j], sem)
                for j in range(CHR)])

    def pbody(t):
        sl = pl.ds(t * L, L)
        cukey_f[sl] = jnp.arange(L, dtype=jnp.int32) + (NN + wid * CH + t * L)
        ceid_f[sl] = jnp.zeros((L,), jnp.int32)
    _vloop(CH // L + 1, pbody)

    for d in descs:
        d.wait()

    def abody(t, off):
        i = t // 8
        j = lax.rem(t, 8)
        sl = pl.ds(j * L, L)
        amb = (b0_v[i, sl] + b1_v[i, sl]) >= 2
        amb_v[i, sl] = amb.astype(jnp.int32)
        cnt = jnp.sum(amb.astype(jnp.int32), axis=0)
        plsc.store_compressed(cukey_f.at[pl.ds(off, L)], ukey_v[i, sl], mask=amb)
        plsc.store_compressed(ceid_f.at[pl.ds(off, L)], eid_v[i, sl], mask=amb)
        return off + cnt
    off = lax.fori_loop(0, CH // L, abody, jnp.int32(0))

    pltpu.sync_copy(amb_v, amb_hbm.at[pl.ds(base, CHR), :])

    nst = (off + 127) // 128

    def kbody(t):
        i = t // 8
        j = lax.rem(t, 8)
        cukey_v[i, pl.ds(j * L, L)] = cukey_f[pl.ds(t * L, L)]
        ceid_v[i, pl.ds(j * L, L)] = ceid_f[pl.ds(t * L, L)]
    _vloop(nst * 8, kbody)

    def drain(jj, carry):
        pltpu.sync_copy(ceid_v.at[jj], darr_hbm.at[cukey_v.at[jj]])
        return carry
    lax.fori_loop(0, nst, drain, 0)


def _k1b(r2, c2, eids2, bloom):
    return pl.kernel(
        _k1b_body,
        out_type=[
            jax.ShapeDtypeStruct((NN + E,), jnp.int32),
            jax.ShapeDtypeStruct((E // 128, 128), jnp.int32),
        ],
        mesh=_mesh,
        compiler_params=_sc_params,
        scratch_types=[
            pltpu.VMEM((CHR, 128), jnp.int32),
            pltpu.VMEM((CHR, 128), jnp.int32),
            pltpu.VMEM((CHR, 128), jnp.int32),
            pltpu.VMEM((CHR, 128), jnp.int32),
            pltpu.VMEM((CHR, 128), jnp.int32),
            pltpu.VMEM((CHR, 128), jnp.int32),
            pltpu.VMEM((CHR, 128), jnp.int32),
            pltpu.VMEM((CHR, 128), jnp.int32),
            pltpu.VMEM((CHR, 128), jnp.int32),
            pltpu.VMEM((CH + L,), jnp.int32),
            pltpu.VMEM((CH + L,), jnp.int32),
            pltpu.VMEM((CHR, 128), jnp.int32),
            pltpu.VMEM((CHR, 128), jnp.int32),
            pltpu.SemaphoreType.DMA,
        ],
    )(r2, c2, eids2, bloom)


def _k2_body(r_hbm, c_hbm, p_hbm, q_hbm, darr_hbm, eids_hbm, amb_hbm,
             w_hbm, spart_hbm,
             r_v, c_v, p_v, q_v, ukey_v, val_v, w_v, eid_v, amb_v,
             zero_v, s_sh, sem):
    cid, sid, wid = _wid()
    base = wid * CHR

    def zbody(i):
        zero_v[pl.ds(i * L, L)] = jnp.zeros((L,), jnp.float32)
    _vloop(4096 // L, zbody)
    pltpu.sync_copy(zero_v, s_sh.at[pl.ds(sid * 4096, 4096)])

    pltpu.sync_copy(r_hbm.at[pl.ds(base, CHR), :], r_v)
    pltpu.sync_copy(c_hbm.at[pl.ds(base, CHR), :], c_v)

    def ubody(t):
        i = t // 8
        j = lax.rem(t, 8)
        rr = r_v[i, pl.ds(j * L, L)]
        cc = c_v[i, pl.ds(j * L, L)]
        ukey_v[i, pl.ds(j * L, L)] = jnp.minimum(rr, cc) * N + jnp.maximum(rr, cc)
    _vloop(CH // L, ubody)

    descs = [pltpu.async_copy(darr_hbm.at[ukey_v.at[j]], w_v.at[j], sem)
             for j in range(CHR)]

    pltpu.sync_copy(p_hbm, p_v)
    pltpu.sync_copy(q_hbm, q_v)
    pltpu.sync_copy(eids_hbm.at[pl.ds(base, CHR), :], eid_v)
    pltpu.sync_copy(amb_hbm.at[pl.ds(base, CHR), :], amb_v)

    def vbody(t):
        i = t // 8
        j = lax.rem(t, 8)
        rr = r_v[i, pl.ds(j * L, L)]
        cc = c_v[i, pl.ds(j * L, L)]
        pv = plsc.load_gather(p_v, [rr])
        qv = plsc.load_gather(q_v, [cc])
        val_v[i, pl.ds(j * L, L)] = 1.0 / (1.0 + jnp.exp(-(pv + qv)))
    _vloop(CH // L, vbody)

    for d in descs:
        d.wait()

    def wbody(t):
        i = t // 8
        j = lax.rem(t, 8)
        sl = pl.ds(j * L, L)
        w_v[i, sl] = jnp.where(amb_v[i, sl] != 0, w_v[i, sl], eid_v[i, sl])
    _vloop(CH // L, wbody)

    pltpu.sync_copy(w_v, w_hbm.at[pl.ds(base, CHR), :])
    plsc.subcore_barrier()
    for j in range(CHR):
        pltpu.sync_copy(val_v.at[j], s_sh.at[w_v.at[j]], add=True)
    plsc.subcore_barrier()
    pltpu.sync_copy(s_sh.at[pl.ds(sid * 4096, 4096)],
                    spart_hbm.at[pl.ds(cid * E + sid * 4096, 4096)])


def _k2(r2, c2, p, q, darr, eids2, amb2):
    return pl.kernel(
        _k2_body,
        out_type=[
            jax.ShapeDtypeStruct((E // 128, 128), jnp.int32),
            jax.ShapeDtypeStruct((NC * E,), jnp.float32),
        ],
        mesh=_mesh,
        compiler_params=_sc_params,
        scratch_types=[
            pltpu.VMEM((CHR, 128), jnp.int32),
            pltpu.VMEM((CHR, 128), jnp.int32),
            pltpu.VMEM((N,), jnp.float32),
            pltpu.VMEM((N,), jnp.float32),
            pltpu.VMEM((CHR, 128), jnp.int32),
            pltpu.VMEM((CHR, 128), jnp.float32),
            pltpu.VMEM((CHR, 128), jnp.int32),
            pltpu.VMEM((CHR, 128), jnp.int32),
            pltpu.VMEM((CHR, 128), jnp.int32),
            pltpu.VMEM((4096,), jnp.float32),
            pltpu.VMEM_SHARED((E,), jnp.float32),
            pltpu.SemaphoreType.DMA,
        ],
    )(r2, c2, p, q, darr, eids2, amb2)


def _k4_body(r_hbm, c_hbm, d_hbm, w_hbm, spart_hbm, nid_hbm, z_hbm,
             outp_hbm, w0p_hbm,
             r_v, c_v, d_v, w_v, w2_v, s0_v, s1_v, coef_f, w0v_f,
             rows_a, rows_b, rows_c, zrows_v, w0z_v, nid_v,
             out_sh, w0_sh, sem, sem2, sem3):
    cid, sid, wid = _wid()
    base = wid * CHR

    def zbody(t):
        i = t // 8
        j = lax.rem(t, 8)
        zrows_v[i, pl.ds(j * L, L)] = jnp.zeros((L,), jnp.float32)
    _vloop(32 * D // L, zbody)

    def z2body(i):
        w0z_v[pl.ds(i * L, L)] = jnp.zeros((L,), jnp.float32)
    _vloop(256 // L, z2body)

    for j in range(8):
        pltpu.sync_copy(zrows_v, out_sh.at[pl.ds(sid * 256 + j * 32, 32), :])
    pltpu.sync_copy(w0z_v, w0_sh.at[pl.ds(sid * 256, 256)])

    pltpu.sync_copy(w_hbm.at[pl.ds(base, CHR), :], w_v)

    def abody(t):
        i = t // 8
        j = lax.rem(t, 8)
        w2_v[i, pl.ds(j * L, L)] = w_v[i, pl.ds(j * L, L)] + E
    _vloop(CH // L, abody)

    descs = ([pltpu.async_copy(spart_hbm.at[w_v.at[j]], s0_v.at[j], sem)
              for j in range(CHR)]
             + [pltpu.async_copy(spart_hbm.at[w2_v.at[j]], s1_v.at[j], sem)
                for j in range(CHR)])

    pltpu.sync_copy(r_hbm.at[pl.ds(base, CHR), :], r_v)
    pltpu.sync_copy(c_hbm.at[pl.ds(base, CHR), :], c_v)
    pltpu.sync_copy(d_hbm.at[pl.ds(base, CHR), :], d_v)
    pltpu.sync_copy(nid_hbm, nid_v)
    for d in descs:
        d.wait()

    def cbody(t):
        i = t // 8
        j = lax.rem(t, 8)
        sl = pl.ds(j * L, L)
        rr = r_v[i, sl]
        cc = c_v[i, sl]
        s = s0_v[i, sl] + s1_v[i, sl]
        co = 0.5 * s * d_v[i, sl]
        co = jnp.where(rr == cc, 0.0, co)
        coef_f[pl.ds(t * L, L)] = co
        w0v_f[pl.ds(t * L, L)] = jnp.where(rr == nid_v[...], co, 0.0)
    _vloop(CH // L, cbody)

    plsc.subcore_barrier()

    def _gather(sub, buf):
        return [pltpu.async_copy(z_hbm.at[c_v.at[sub * SUBR + j]],
                                 buf.at[pl.ds(j * 128, 128), :], sem2)
                for j in range(SUBR)]

    def _scale(sub, buf):
        def sbody(h):
            i = h * 4
            cbs = [plsc.load_gather(
                coef_f, [jnp.zeros((L,), jnp.int32) + (sub * SUB + i + k)])
                for k in range(4)]
            for k in range(4):
                for jj in range(D // L):
                    sl = pl.ds(jj * L, L)
                    buf[i + k, sl] = buf[i + k, sl] * cbs[k]
        _vloop(SUB // 4, sbody)

    def _scatter(sub, buf):
        return [pltpu.async_copy(buf.at[pl.ds(j * 128, 128), :],
                                 out_sh.at[r_v.at[sub * SUBR + j]], sem3,
                                 add=True)
                for j in range(SUBR)]

    bufs = [rows_a, rows_b, rows_c]
    gd = {}
    sd = {}
    gd[0] = _gather(0, bufs[0])
    for sub in range(NSUB):
        if sub + 1 < NSUB:
            if sub + 1 >= 3:
                for d in sd[sub + 1 - 3]:
                    d.wait()
            gd[sub + 1] = _gather(sub + 1, bufs[(sub + 1) % 3])
        for d in gd[sub]:
            d.wait()
        _scale(sub, bufs[sub % 3])
        sd[sub] = _scatter(sub, bufs[sub % 3])
    for sub in range(max(0, NSUB - 3), NSUB):
        for d in sd[sub]:
            d.wait()

    for j in range(CHR):
        pltpu.sync_copy(w0v_f.at[pl.ds(j * 128, 128)],
                        w0_sh.at[c_v.at[j]], add=True)
    plsc.subcore_barrier()

    pltpu.sync_copy(out_sh.at[pl.ds(sid * 256, 256), :],
                    outp_hbm.at[pl.ds(cid * N + sid * 256, 256), :])
    pltpu.sync_copy(w0_sh.at[pl.ds(sid * 256, 256)],
                    w0p_hbm.at[pl.ds(cid * N + sid * 256, 256)])


def _k4(r2, c2, data2, w2d, spart, nid, z):
    return pl.kernel(
        _k4_body,
        out_type=[
            jax.ShapeDtypeStruct((NC * N, D), jnp.float32),
            jax.ShapeDtypeStruct((NC * N,), jnp.float32),
        ],
        mesh=_mesh,
        compiler_params=_sc_params,
        scratch_types=[
            pltpu.VMEM((CHR, 128), jnp.int32),
            pltpu.VMEM((CHR, 128), jnp.int32),
            pltpu.VMEM((CHR, 128), jnp.float32),
            pltpu.VMEM((CHR, 128), jnp.int32),
            pltpu.VMEM((CHR, 128), jnp.int32),
            pltpu.VMEM((CHR, 128), jnp.float32),
            pltpu.VMEM((CHR, 128), jnp.float32),
            pltpu.VMEM((CH,), jnp.float32),
            pltpu.VMEM((CH,), jnp.float32),
            pltpu.VMEM((SUB, D), jnp.float32),
            pltpu.VMEM((SUB, D), jnp.float32),
            pltpu.VMEM((SUB, D), jnp.float32),
            pltpu.VMEM((32, D), jnp.float32),
            pltpu.VMEM((256,), jnp.float32),
            pltpu.VMEM((L,), jnp.int32),
            pltpu.VMEM_SHARED((N, D), jnp.float32),
            pltpu.VMEM_SHARED((N,), jnp.float32),
            pltpu.SemaphoreType.DMA,
            pltpu.SemaphoreType.DMA,
            pltpu.SemaphoreType.DMA,
        ],
    )(r2, c2, data2, w2d, spart, nid, z)


def _k5_body(out0_ref, out1_ref, w0a_ref, w0b_ref, w2_ref, res_ref, acc_ref):
    i = pl.program_id(0)
    h = jnp.maximum(out0_ref[...] + out1_ref[...], 0.0)
    wv = w0a_ref[...] + w0b_ref[...]
    contrib = jnp.sum(wv * h, axis=0, keepdims=True)

    @pl.when(i == 0)
    def _():
        acc_ref[...] = contrib

    @pl.when(i != 0)
    def _():
        acc_ref[...] = acc_ref[...] + contrib

    @pl.when(i == NBLK - 1)
    def _():
        r16 = lax.dot_general(
            acc_ref[...], w2_ref[...], (((1,), (0,)), ((), ())),
            preferred_element_type=jnp.float32)
        m = jnp.max(r16)
        e = jnp.exp(r16 - m)
        res_ref[...] = e / jnp.sum(e)


def _k5(outp, w0p2, W2):
    return pl.pallas_call(
        _k5_body,
        grid=(NBLK,),
        in_specs=[
            pl.BlockSpec((ROWBLK, D), lambda i: (i, 0)),
            pl.BlockSpec((ROWBLK, D), lambda i: (i + NBLK, 0)),
            pl.BlockSpec((ROWBLK, 1), lambda i: (i, 0)),
            pl.BlockSpec((ROWBLK, 1), lambda i: (i + NBLK, 0)),
            pl.BlockSpec((D, C), lambda i: (0, 0)),
        ],
        out_specs=pl.BlockSpec((1, C), lambda i: (0, 0)),
        out_shape=jax.ShapeDtypeStruct((1, C), jnp.float32),
        scratch_shapes=[pltpu.VMEM((1, D), jnp.float32)],
    )(outp, outp, w0p2, w0p2, W2)


def kernel(x, embed, adj_row, adj_col, adj_data, nodeid, sub_new_edge_index,
           tmp, W_mask, b_mask, W1, W2):
    nodeid = jnp.asarray(nodeid, jnp.int32)
    c0 = (embed[nodeid] @ W_mask[2 * D:, 0] + b_mask[0]).reshape(1, 1)
    p2, q2, z = _k0(c0, embed, x, W_mask, W1)
    p, q = p2.reshape(N), q2.reshape(N)

    r2 = adj_row.astype(jnp.int32).reshape(E // 128, 128)
    c2 = adj_col.astype(jnp.int32).reshape(E // 128, 128)
    data2 = adj_data.astype(jnp.float32).reshape(E // 128, 128)
    nid = jnp.broadcast_to(nodeid, (L,)).astype(jnp.int32)

    eids2 = jnp.arange(E, dtype=jnp.int32).reshape(E // 128, 128)
    bloom = _k1a(r2, c2)
    darr, amb2 = _k1b(r2, c2, eids2, bloom)
    w2d, spart = _k2(r2, c2, p, q, darr, eids2, amb2)
    outp, w0p = _k4(r2, c2, data2, w2d, spart, nid, z)

    res = _k5(outp, w0p.reshape(NC * N, 1), W2)
    return res.reshape(C)


# speedup vs baseline: 1.0083x; 1.0083x over previous
import functools

import jax
import jax.numpy as jnp
from jax import lax
from jax.experimental import pallas as pl
from jax.experimental.pallas import tpu as pltpu
from jax.experimental.pallas import tpu_sc as plsc

N = 4096
E = 65536
D = 128
C = 16
NN = N * N

NC = 2
NS = 16
NW = NC * NS
L = 16

CH = E // NW
CHR = CH // 128
SUB = 128
NSUB = CH // SUB
SUBR = SUB // 128

ROWBLK = 256
NBLK = N // ROWBLK

_mesh = plsc.VectorSubcoreMesh(core_axis_name="c", subcore_axis_name="s")
_sc_params = pltpu.CompilerParams(needs_layout_passes=False)


def _wid():
    cid = lax.axis_index("c")
    sid = lax.axis_index("s")
    return cid, sid, cid * NS + sid


def _vloop(n16, body):
    def step(i, carry):
        body(i)
        return carry
    lax.fori_loop(0, n16, step, 0)


def _k0_body(c0_ref, embed_ref, x_ref, wm_ref, w1_ref, p_ref, q_ref, z_ref):
    emb = embed_ref[...]
    p_ref[...] = lax.dot_general(
        emb, wm_ref[0:D, :], (((1,), (0,)), ((), ())),
        preferred_element_type=jnp.float32)
    q_ref[...] = lax.dot_general(
        emb, wm_ref[D:2 * D, :], (((1,), (0,)), ((), ())),
        preferred_element_type=jnp.float32) + c0_ref[0, 0]
    z_ref[...] = lax.dot_general(
        x_ref[...], w1_ref[...], (((1,), (0,)), ((), ())),
        preferred_element_type=jnp.float32)


def _k0(c0, embed, x, W_mask, W1):
    return pl.pallas_call(
        _k0_body,
        grid=(NBLK,),
        in_specs=[
            pl.BlockSpec(memory_space=pltpu.SMEM),
            pl.BlockSpec((ROWBLK, D), lambda i: (i, 0)),
            pl.BlockSpec((ROWBLK, D), lambda i: (i, 0)),
            pl.BlockSpec((3 * D, 1), lambda i: (0, 0)),
            pl.BlockSpec((D, D), lambda i: (0, 0)),
        ],
        out_specs=[
            pl.BlockSpec((ROWBLK, 1), lambda i: (i, 0)),
            pl.BlockSpec((ROWBLK, 1), lambda i: (i, 0)),
            pl.BlockSpec((ROWBLK, D), lambda i: (i, 0)),
        ],
        out_shape=[
            jax.ShapeDtypeStruct((N, 1), jnp.float32),
            jax.ShapeDtypeStruct((N, 1), jnp.float32),
            jax.ShapeDtypeStruct((N, D), jnp.float32),
        ],
    )(c0, embed, x, W_mask, W1)


BHASH = 1 << 20
BSLICE = BHASH // NS


def _hash(uk):
    return jnp.bitwise_and(jnp.bitwise_xor(uk, uk >> 11), BHASH - 1)


def _k1a_body(r_hbm, c_hbm, bloom_hbm,
              r_v, c_v, h_v, ones_v, zero_v, bloom_sh, sem):
    cid, sid, wid = _wid()
    base = wid * CHR

    def zbody(i):
        zero_v[pl.ds(i * L, L)] = jnp.zeros((L,), jnp.int32)
    _vloop(4096 // L, zbody)
    for j in range(BSLICE // 4096):
        pltpu.sync_copy(zero_v,
                        bloom_sh.at[pl.ds(sid * BSLICE + j * 4096, 4096)])

    def obody(t):
        i = t // 8
        j = lax.rem(t, 8)
        ones_v[i, pl.ds(j * L, L)] = jnp.zeros((L,), jnp.int32) + 1
    _vloop(CH // L, obody)

    pltpu.sync_copy(r_hbm.at[pl.ds(base, CHR), :], r_v)
    pltpu.sync_copy(c_hbm.at[pl.ds(base, CHR), :], c_v)

    def body(t):
        i = t // 8
        j = lax.rem(t, 8)
        rr = r_v[i, pl.ds(j * L, L)]
        cc = c_v[i, pl.ds(j * L, L)]
        uk = jnp.minimum(rr, cc) * N + jnp.maximum(rr, cc)
        h_v[i, pl.ds(j * L, L)] = _hash(uk)
    _vloop(CH // L, body)

    plsc.subcore_barrier()
    for j in range(CHR):
        pltpu.sync_copy(ones_v.at[j], bloom_sh.at[h_v.at[j]], add=True)
    plsc.subcore_barrier()
    pltpu.sync_copy(bloom_sh.at[pl.ds(sid * BSLICE, BSLICE)],
                    bloom_hbm.at[pl.ds(cid * BHASH + sid * BSLICE, BSLICE)])


def _k1a(r2, c2):
    return pl.kernel(
        _k1a_body,
        out_type=jax.ShapeDtypeStruct((NC * BHASH,), jnp.int32),
        mesh=_mesh,
        compiler_params=_sc_params,
        scratch_types=[
            pltpu.VMEM((CHR, 128), jnp.int32),
            pltpu.VMEM((CHR, 128), jnp.int32),
            pltpu.VMEM((CHR, 128), jnp.int32),
            pltpu.VMEM((CHR, 128), jnp.int32),
            pltpu.VMEM((4096,), jnp.int32),
            pltpu.VMEM_SHARED((BHASH,), jnp.int32),
            pltpu.SemaphoreType.DMA,
        ],
    )(r2, c2)


def _k1b_body(r_hbm, c_hbm, eids_hbm, bloom_hbm,
              darr_hbm, amb_hbm,
              r_v, c_v, eid_v, ukey_v, h_v, h2_v, b0_v, b1_v, amb_v,
              cukey_f, ceid_f, cukey_v, ceid_v, sem):
    cid, sid, wid = _wid()
    base = wid * CHR

    pltpu.sync_copy(r_hbm.at[pl.ds(base, CHR), :], r_v)
    pltpu.sync_copy(c_hbm.at[pl.ds(base, CHR), :], c_v)
    pltpu.sync_copy(eids_hbm.at[pl.ds(base, CHR), :], eid_v)

    def body(t):
        i = t // 8
        j = lax.rem(t, 8)
        rr = r_v[i, pl.ds(j * L, L)]
        cc = c_v[i, pl.ds(j * L, L)]
        uk = jnp.minimum(rr, cc) * N + jnp.maximum(rr, cc)
        h = _hash(uk)
        ukey_v[i, pl.ds(j * L, L)] = uk
        h_v[i, pl.ds(j * L, L)] = h
        h2_v[i, pl.ds(j * L, L)] = h + BHASH
    _vloop(CH // L, body)

    descs = ([pltpu.async_copy(bloom_hbm.at[h_v.at[j]], b0_v.at[j], sem)
              for j in range(CHR)]
             + [pltpu.async_copy(bloom_hbm.at[h2_v.at[j]], b1_v.at[j], sem)
                for j in range(CHR)])

    def pbody(t):
        sl = pl.ds(t * L, L)
        cukey_f[sl] = jnp.arange(L, dtype=jnp.int32) + (NN + wid * CH + t * L)
        ceid_f[sl] = jnp.zeros((L,), jnp.int32)
    _vloop(CH // L + 1, pbody)

    for d in descs:
        d.wait()

    def abody(t, off):
        i = t // 8
        j = lax.rem(t, 8)
        sl = pl.ds(j * L, L)
        amb = (b0_v[i, sl] + b1_v[i, sl]) >= 2
        amb_v[i, sl] = amb.astype(jnp.int32)
        cnt = jnp.sum(amb.astype(jnp.int32), axis=0)
        plsc.store_compressed(cukey_f.at[pl.ds(off, L)], ukey_v[i, sl], mask=amb)
        plsc.store_compressed(ceid_f.at[pl.ds(off, L)], eid_v[i, sl], mask=amb)
        return off + cnt
    off = lax.fori_loop(0, CH // L, abody, jnp.int32(0))

    pltpu.sync_copy(amb_v, amb_hbm.at[pl.ds(base, CHR), :])

    nst = (off + 127) // 128

    def kbody(t):
        i = t // 8
        j = lax.rem(t, 8)
        cukey_v[i, pl.ds(j * L, L)] = cukey_f[pl.ds(t * L, L)]
        ceid_v[i, pl.ds(j * L, L)] = ceid_f[pl.ds(t * L, L)]
    _vloop(nst * 8, kbody)

    def drain(jj, carry):
        pltpu.sync_copy(ceid_v.at[jj], darr_hbm.at[cukey_v.at[jj]])
        return carry
    lax.fori_loop(0, nst, drain, 0)


def _k1b(r2, c2, eids2, bloom):
    return pl.kernel(
        _k1b_body,
        out_type=[
            jax.ShapeDtypeStruct((NN + E,), jnp.int32),
            jax.ShapeDtypeStruct((E // 128, 128), jnp.int32),
        ],
        mesh=_mesh,
        compiler_params=_sc_params,
        scratch_types=[
            pltpu.VMEM((CHR, 128), jnp.int32),
            pltpu.VMEM((CHR, 128), jnp.int32),
            pltpu.VMEM((CHR, 128), jnp.int32),
            pltpu.VMEM((CHR, 128), jnp.int32),
            pltpu.VMEM((CHR, 128), jnp.int32),
            pltpu.VMEM((CHR, 128), jnp.int32),
            pltpu.VMEM((CHR, 128), jnp.int32),
            pltpu.VMEM((CHR, 128), jnp.int32),
            pltpu.VMEM((CHR, 128), jnp.int32),
            pltpu.VMEM((CH + L,), jnp.int32),
            pltpu.VMEM((CH + L,), jnp.int32),
            pltpu.VMEM((CHR, 128), jnp.int32),
            pltpu.VMEM((CHR, 128), jnp.int32),
            pltpu.SemaphoreType.DMA,
        ],
    )(r2, c2, eids2, bloom)


def _k2_body(r_hbm, c_hbm, p_hbm, q_hbm, darr_hbm, eids_hbm, amb_hbm,
             w_hbm, spart_hbm,
             r_v, c_v, p_v, q_v, ukey_v, val_v, w_v, eid_v, amb_v,
             zero_v, s_sh, sem):
    cid, sid, wid = _wid()
    base = wid * CHR

    def zbody(i):
        zero_v[pl.ds(i * L, L)] = jnp.zeros((L,), jnp.float32)
    _vloop(4096 // L, zbody)
    pltpu.sync_copy(zero_v, s_sh.at[pl.ds(sid * 4096, 4096)])

    pltpu.sync_copy(r_hbm.at[pl.ds(base, CHR), :], r_v)
    pltpu.sync_copy(c_hbm.at[pl.ds(base, CHR), :], c_v)

    def ubody(t):
        i = t // 8
        j = lax.rem(t, 8)
        rr = r_v[i, pl.ds(j * L, L)]
        cc = c_v[i, pl.ds(j * L, L)]
        ukey_v[i, pl.ds(j * L, L)] = jnp.minimum(rr, cc) * N + jnp.maximum(rr, cc)
    _vloop(CH // L, ubody)

    descs = [pltpu.async_copy(darr_hbm.at[ukey_v.at[j]], w_v.at[j], sem)
             for j in range(CHR)]

    pltpu.sync_copy(p_hbm, p_v)
    pltpu.sync_copy(q_hbm, q_v)
    pltpu.sync_copy(eids_hbm.at[pl.ds(base, CHR), :], eid_v)
    pltpu.sync_copy(amb_hbm.at[pl.ds(base, CHR), :], amb_v)

    def vbody(t):
        i = t // 8
        j = lax.rem(t, 8)
        rr = r_v[i, pl.ds(j * L, L)]
        cc = c_v[i, pl.ds(j * L, L)]
        pv = plsc.load_gather(p_v, [rr])
        qv = plsc.load_gather(q_v, [cc])
        val_v[i, pl.ds(j * L, L)] = 1.0 / (1.0 + jnp.exp(-(pv + qv)))
    _vloop(CH // L, vbody)

    for d in descs:
        d.wait()

    def wbody(t):
        i = t // 8
        j = lax.rem(t, 8)
        sl = pl.ds(j * L, L)
        w_v[i, sl] = jnp.where(amb_v[i, sl] != 0, w_v[i, sl], eid_v[i, sl])
    _vloop(CH // L, wbody)

    pltpu.sync_copy(w_v, w_hbm.at[pl.ds(base, CHR), :])
    plsc.subcore_barrier()
    for j in range(CHR):
        pltpu.sync_copy(val_v.at[j], s_sh.at[w_v.at[j]], add=True)
    plsc.subcore_barrier()
    pltpu.sync_copy(s_sh.at[pl.ds(sid * 4096, 4096)],
                    spart_hbm.at[pl.ds(cid * E + sid * 4096, 4096)])


def _k2(r2, c2, p, q, darr, eids2, amb2):
    return pl.kernel(
        _k2_body,
        out_type=[
            jax.ShapeDtypeStruct((E // 128, 128), jnp.int32),
            jax.ShapeDtypeStruct((NC * E,), jnp.float32),
        ],
        mesh=_mesh,
        compiler_params=_sc_params,
        scratch_types=[
            pltpu.VMEM((CHR, 128), jnp.int32),
            pltpu.VMEM((CHR, 128), jnp.int32),
            pltpu.VMEM((N,), jnp.float32),
            pltpu.VMEM((N,), jnp.float32),
            pltpu.VMEM((CHR, 128), jnp.int32),
            pltpu.VMEM((CHR, 128), jnp.float32),
            pltpu.VMEM((CHR, 128), jnp.int32),
            pltpu.VMEM((CHR, 128), jnp.int32),
            pltpu.VMEM((CHR, 128), jnp.int32),
            pltpu.VMEM((4096,), jnp.float32),
            pltpu.VMEM_SHARED((E,), jnp.float32),
            pltpu.SemaphoreType.DMA,
        ],
    )(r2, c2, p, q, darr, eids2, amb2)


def _k4_body(r_hbm, c_hbm, d_hbm, w_hbm, spart_hbm, nid_hbm, z_hbm,
             outp_hbm, w0p_hbm,
             r_v, c_v, d_v, w_v, w2_v, s0_v, s1_v, coef_f, w0v_f,
             rows_a, rows_b, rows_c, zrows_v, w0z_v, nid_v,
             out_sh, w0_sh, sem, sem2, sem3):
    cid, sid, wid = _wid()
    base = wid * CHR

    def zbody(t):
        i = t // 8
        j = lax.rem(t, 8)
        zrows_v[i, pl.ds(j * L, L)] = jnp.zeros((L,), jnp.float32)
    _vloop(32 * D // L, zbody)

    def z2body(i):
        w0z_v[pl.ds(i * L, L)] = jnp.zeros((L,), jnp.float32)
    _vloop(256 // L, z2body)

    for j in range(8):
        pltpu.sync_copy(zrows_v, out_sh.at[pl.ds(sid * 256 + j * 32, 32), :])
    pltpu.sync_copy(w0z_v, w0_sh.at[pl.ds(sid * 256, 256)])

    pltpu.sync_copy(w_hbm.at[pl.ds(base, CHR), :], w_v)

    def abody(t):
        i = t // 8
        j = lax.rem(t, 8)
        w2_v[i, pl.ds(j * L, L)] = w_v[i, pl.ds(j * L, L)] + E
    _vloop(CH // L, abody)

    descs = ([pltpu.async_copy(spart_hbm.at[w_v.at[j]], s0_v.at[j], sem)
              for j in range(CHR)]
             + [pltpu.async_copy(spart_hbm.at[w2_v.at[j]], s1_v.at[j], sem)
                for j in range(CHR)])

    pltpu.sync_copy(r_hbm.at[pl.ds(base, CHR), :], r_v)
    pltpu.sync_copy(c_hbm.at[pl.ds(base, CHR), :], c_v)
    pltpu.sync_copy(d_hbm.at[pl.ds(base, CHR), :], d_v)
    pltpu.sync_copy(nid_hbm, nid_v)

    def _gather(sub, buf):
        return [pltpu.async_copy(z_hbm.at[c_v.at[sub * SUBR + j]],
                                 buf.at[pl.ds(j * 128, 128), :], sem2)
                for j in range(SUBR)]

    gd = {}
    gd[0] = _gather(0, rows_a)
    gd[1] = _gather(1, rows_b)

    for d in descs:
        d.wait()

    def cbody(t):
        i = t // 8
        j = lax.rem(t, 8)
        sl = pl.ds(j * L, L)
        rr = r_v[i, sl]
        cc = c_v[i, sl]
        s = s0_v[i, sl] + s1_v[i, sl]
        co = 0.5 * s * d_v[i, sl]
        co = jnp.where(rr == cc, 0.0, co)
        coef_f[pl.ds(t * L, L)] = co
        w0v_f[pl.ds(t * L, L)] = jnp.where(rr == nid_v[...], co, 0.0)
    _vloop(CH // L, cbody)

    plsc.subcore_barrier()

    def _scale(sub, buf):
        def sbody(h):
            i = h * 4
            cbs = [plsc.load_gather(
                coef_f, [jnp.zeros((L,), jnp.int32) + (sub * SUB + i + k)])
                for k in range(4)]
            for k in range(4):
                for jj in range(D // L):
                    sl = pl.ds(jj * L, L)
                    buf[i + k, sl] = buf[i + k, sl] * cbs[k]
        _vloop(SUB // 4, sbody)

    def _scatter(sub, buf):
        return [pltpu.async_copy(buf.at[pl.ds(j * 128, 128), :],
                                 out_sh.at[r_v.at[sub * SUBR + j]], sem3,
                                 add=True)
                for j in range(SUBR)]

    bufs = [rows_a, rows_b, rows_c]
    sd = {}
    for sub in range(NSUB):
        for d in gd[sub]:
            d.wait()
        _scale(sub, bufs[sub % 3])
        sd[sub] = _scatter(sub, bufs[sub % 3])
        if sub + 2 < NSUB:
            if sub - 1 >= 0:
                for d in sd[sub - 1]:
                    d.wait()
            gd[sub + 2] = _gather(sub + 2, bufs[(sub + 2) % 3])
    for sub in range(max(0, NSUB - 3), NSUB):
        for d in sd[sub]:
            d.wait()

    for j in range(CHR):
        pltpu.sync_copy(w0v_f.at[pl.ds(j * 128, 128)],
                        w0_sh.at[c_v.at[j]], add=True)
    plsc.subcore_barrier()

    pltpu.sync_copy(out_sh.at[pl.ds(sid * 256, 256), :],
                    outp_hbm.at[pl.ds(cid * N + sid * 256, 256), :])
    pltpu.sync_copy(w0_sh.at[pl.ds(sid * 256, 256)],
                    w0p_hbm.at[pl.ds(cid * N + sid * 256, 256)])


def _k4(r2, c2, data2, w2d, spart, nid, z):
    return pl.kernel(
        _k4_body,
        out_type=[
            jax.ShapeDtypeStruct((NC * N, D), jnp.float32),
            jax.ShapeDtypeStruct((NC * N,), jnp.float32),
        ],
        mesh=_mesh,
        compiler_params=_sc_params,
        scratch_types=[
            pltpu.VMEM((CHR, 128), jnp.int32),
            pltpu.VMEM((CHR, 128), jnp.int32),
            pltpu.VMEM((CHR, 128), jnp.float32),
            pltpu.VMEM((CHR, 128), jnp.int32),
            pltpu.VMEM((CHR, 128), jnp.int32),
            pltpu.VMEM((CHR, 128), jnp.float32),
            pltpu.VMEM((CHR, 128), jnp.float32),
            pltpu.VMEM((CH,), jnp.float32),
            pltpu.VMEM((CH,), jnp.float32),
            pltpu.VMEM((SUB, D), jnp.float32),
            pltpu.VMEM((SUB, D), jnp.float32),
            pltpu.VMEM((SUB, D), jnp.float32),
            pltpu.VMEM((32, D), jnp.float32),
            pltpu.VMEM((256,), jnp.float32),
            pltpu.VMEM((L,), jnp.int32),
            pltpu.VMEM_SHARED((N, D), jnp.float32),
            pltpu.VMEM_SHARED((N,), jnp.float32),
            pltpu.SemaphoreType.DMA,
            pltpu.SemaphoreType.DMA,
            pltpu.SemaphoreType.DMA,
        ],
    )(r2, c2, data2, w2d, spart, nid, z)


def _k5_body(out0_ref, out1_ref, w0a_ref, w0b_ref, w2_ref, res_ref, acc_ref):
    i = pl.program_id(0)
    h = jnp.maximum(out0_ref[...] + out1_ref[...], 0.0)
    wv = w0a_ref[...] + w0b_ref[...]
    contrib = jnp.sum(wv * h, axis=0, keepdims=True)

    @pl.when(i == 0)
    def _():
        acc_ref[...] = contrib

    @pl.when(i != 0)
    def _():
        acc_ref[...] = acc_ref[...] + contrib

    @pl.when(i == NBLK - 1)
    def _():
        r16 = lax.dot_general(
            acc_ref[...], w2_ref[...], (((1,), (0,)), ((), ())),
            preferred_element_type=jnp.float32)
        m = jnp.max(r16)
        e = jnp.exp(r16 - m)
        res_ref[...] = e / jnp.sum(e)


def _k5(outp, w0p2, W2):
    return pl.pallas_call(
        _k5_body,
        grid=(NBLK,),
        in_specs=[
            pl.BlockSpec((ROWBLK, D), lambda i: (i, 0)),
            pl.BlockSpec((ROWBLK, D), lambda i: (i + NBLK, 0)),
            pl.BlockSpec((ROWBLK, 1), lambda i: (i, 0)),
            pl.BlockSpec((ROWBLK, 1), lambda i: (i + NBLK, 0)),
            pl.BlockSpec((D, C), lambda i: (0, 0)),
        ],
        out_specs=pl.BlockSpec((1, C), lambda i: (0, 0)),
        out_shape=jax.ShapeDtypeStruct((1, C), jnp.float32),
        scratch_shapes=[pltpu.VMEM((1, D), jnp.float32)],
    )(outp, outp, w0p2, w0p2, W2)


def kernel(x, embed, adj_row, adj_col, adj_data, nodeid, sub_new_edge_index,
           tmp, W_mask, b_mask, W1, W2):
    nodeid = jnp.asarray(nodeid, jnp.int32)
    c0 = (embed[nodeid] @ W_mask[2 * D:, 0] + b_mask[0]).reshape(1, 1)
    p2, q2, z = _k0(c0, embed, x, W_mask, W1)
    p, q = p2.reshape(N), q2.reshape(N)

    r2 = adj_row.astype(jnp.int32).reshape(E // 128, 128)
    c2 = adj_col.astype(jnp.int32).reshape(E // 128, 128)
    data2 = adj_data.astype(jnp.float32).reshape(E // 128, 128)
    nid = jnp.broadcast_to(nodeid, (L,)).astype(jnp.int32)

    eids2 = jnp.arange(E, dtype=jnp.int32).reshape(E // 128, 128)
    bloom = _k1a(r2, c2)
    darr, amb2 = _k1b(r2, c2, eids2, bloom)
    w2d, spart = _k2(r2, c2, p, q, darr, eids2, amb2)
    outp, w0p = _k4(r2, c2, data2, w2d, spart, nid, z)

    res = _k5(outp, w0p.reshape(NC * N, 1), W2)
    return res.reshape(C)


# speedup vs baseline: 1.0099x; 1.0016x over previous
import jax
import jax.numpy as jnp
from jax import lax
from jax.experimental import pallas as pl
from jax.experimental.pallas import tpu as pltpu
from jax.experimental.pallas import tpu_sc as plsc

N = 4096
E = 65536
D = 128
C = 16
NN = N * N

NC = 2
NS = 16
NW = NC * NS
L = 16

CH = E // NW
CHR = CH // 128
SUB = 128
NSUB = CH // SUB
SUBR = SUB // 128

ROWBLK = 256
NBLK = N // ROWBLK

_mesh = plsc.VectorSubcoreMesh(core_axis_name="c", subcore_axis_name="s")
_sc_params = pltpu.CompilerParams(needs_layout_passes=False)


def _wid():
    cid = lax.axis_index("c")
    sid = lax.axis_index("s")
    return cid, sid, cid * NS + sid


def _vloop(n16, body):
    def step(i, carry):
        body(i)
        return carry
    lax.fori_loop(0, n16, step, 0)


def _k0_body(c0_ref, embed_ref, x_ref, wm_ref, w1_ref, p_ref, q_ref, z_ref):
    emb = embed_ref[...]
    p_ref[...] = lax.dot_general(
        emb, wm_ref[0:D, :], (((1,), (0,)), ((), ())),
        preferred_element_type=jnp.float32)
    q_ref[...] = lax.dot_general(
        emb, wm_ref[D:2 * D, :], (((1,), (0,)), ((), ())),
        preferred_element_type=jnp.float32) + c0_ref[0, 0]
    z_ref[...] = lax.dot_general(
        x_ref[...], w1_ref[...], (((1,), (0,)), ((), ())),
        preferred_element_type=jnp.float32)


def _k0(c0, embed, x, W_mask, W1):
    return pl.pallas_call(
        _k0_body,
        grid=(NBLK,),
        in_specs=[
            pl.BlockSpec(memory_space=pltpu.SMEM),
            pl.BlockSpec((ROWBLK, D), lambda i: (i, 0)),
            pl.BlockSpec((ROWBLK, D), lambda i: (i, 0)),
            pl.BlockSpec((3 * D, 1), lambda i: (0, 0)),
            pl.BlockSpec((D, D), lambda i: (0, 0)),
        ],
        out_specs=[
            pl.BlockSpec((ROWBLK, 1), lambda i: (i, 0)),
            pl.BlockSpec((ROWBLK, 1), lambda i: (i, 0)),
            pl.BlockSpec((ROWBLK, D), lambda i: (i, 0)),
        ],
        out_shape=[
            jax.ShapeDtypeStruct((N, 1), jnp.float32),
            jax.ShapeDtypeStruct((N, 1), jnp.float32),
            jax.ShapeDtypeStruct((N, D), jnp.float32),
        ],
    )(c0, embed, x, W_mask, W1)


BHASH = 1 << 20
BSLICE = BHASH // NS


def _hash(uk):
    return jnp.bitwise_and(jnp.bitwise_xor(uk, uk >> 11), BHASH - 1)


def _k1a_body(r_hbm, c_hbm, bloom_hbm,
              r_v, c_v, h_v, ones_v, zero_v, bloom_sh, sem):
    cid, sid, wid = _wid()
    base = wid * CHR

    def zbody(i):
        zero_v[pl.ds(i * L, L)] = jnp.zeros((L,), jnp.int32)
    _vloop(4096 // L, zbody)
    for j in range(BSLICE // 4096):
        pltpu.sync_copy(zero_v,
                        bloom_sh.at[pl.ds(sid * BSLICE + j * 4096, 4096)])

    def obody(t):
        i = t // 8
        j = lax.rem(t, 8)
        ones_v[i, pl.ds(j * L, L)] = jnp.zeros((L,), jnp.int32) + 1
    _vloop(CH // L, obody)

    pltpu.sync_copy(r_hbm.at[pl.ds(base, CHR), :], r_v)
    pltpu.sync_copy(c_hbm.at[pl.ds(base, CHR), :], c_v)

    def body(t):
        i = t // 8
        j = lax.rem(t, 8)
        rr = r_v[i, pl.ds(j * L, L)]
        cc = c_v[i, pl.ds(j * L, L)]
        uk = jnp.minimum(rr, cc) * N + jnp.maximum(rr, cc)
        h_v[i, pl.ds(j * L, L)] = _hash(uk)
    _vloop(CH // L, body)

    plsc.subcore_barrier()
    for j in range(CHR):
        pltpu.sync_copy(ones_v.at[j], bloom_sh.at[h_v.at[j]], add=True)
    plsc.subcore_barrier()
    pltpu.sync_copy(bloom_sh.at[pl.ds(sid * BSLICE, BSLICE)],
                    bloom_hbm.at[pl.ds(cid * BHASH + sid * BSLICE, BSLICE)])


def _k1a(r2, c2):
    return pl.kernel(
        _k1a_body,
        out_type=jax.ShapeDtypeStruct((NC * BHASH,), jnp.int32),
        mesh=_mesh,
        compiler_params=_sc_params,
        scratch_types=[
            pltpu.VMEM((CHR, 128), jnp.int32),
            pltpu.VMEM((CHR, 128), jnp.int32),
            pltpu.VMEM((CHR, 128), jnp.int32),
            pltpu.VMEM((CHR, 128), jnp.int32),
            pltpu.VMEM((4096,), jnp.int32),
            pltpu.VMEM_SHARED((BHASH,), jnp.int32),
            pltpu.SemaphoreType.DMA,
        ],
    )(r2, c2)


def _k1b_body(r_hbm, c_hbm, eids_hbm, bloom_hbm,
              darr_hbm, amb_hbm,
              r_v, c_v, eid_v, ukey_v, h_v, h2_v, b0_v, b1_v, amb_v,
              cukey_f, ceid_f, cukey_v, ceid_v, sem):
    cid, sid, wid = _wid()
    base = wid * CHR

    pltpu.sync_copy(r_hbm.at[pl.ds(base, CHR), :], r_v)
    pltpu.sync_copy(c_hbm.at[pl.ds(base, CHR), :], c_v)
    pltpu.sync_copy(eids_hbm.at[pl.ds(base, CHR), :], eid_v)

    def body(t):
        i = t // 8
        j = lax.rem(t, 8)
        rr = r_v[i, pl.ds(j * L, L)]
        cc = c_v[i, pl.ds(j * L, L)]
        uk = jnp.minimum(rr, cc) * N + jnp.maximum(rr, cc)
        h = _hash(uk)
        ukey_v[i, pl.ds(j * L, L)] = uk
        h_v[i, pl.ds(j * L, L)] = h
        h2_v[i, pl.ds(j * L, L)] = h + BHASH
    _vloop(CH // L, body)

    descs = ([pltpu.async_copy(bloom_hbm.at[h_v.at[j]], b0_v.at[j], sem)
              for j in range(CHR)]
             + [pltpu.async_copy(bloom_hbm.at[h2_v.at[j]], b1_v.at[j], sem)
                for j in range(CHR)])

    def pbody(t):
        sl = pl.ds(t * L, L)
        cukey_f[sl] = jnp.arange(L, dtype=jnp.int32) + (NN + wid * CH + t * L)
        ceid_f[sl] = jnp.zeros((L,), jnp.int32)
    _vloop(CH // L + 1, pbody)

    for d in descs:
        d.wait()

    def abody(t, off):
        i = t // 8
        j = lax.rem(t, 8)
        sl = pl.ds(j * L, L)
        amb = (b0_v[i, sl] + b1_v[i, sl]) >= 2
        amb_v[i, sl] = amb.astype(jnp.int32)
        cnt = jnp.sum(amb.astype(jnp.int32), axis=0)
        plsc.store_compressed(cukey_f.at[pl.ds(off, L)], ukey_v[i, sl], mask=amb)
        plsc.store_compressed(ceid_f.at[pl.ds(off, L)], eid_v[i, sl], mask=amb)
        return off + cnt
    off = lax.fori_loop(0, CH // L, abody, jnp.int32(0))

    pltpu.sync_copy(amb_v, amb_hbm.at[pl.ds(base, CHR), :])

    nst = (off + 127) // 128

    def kbody(t):
        i = t // 8
        j = lax.rem(t, 8)
        cukey_v[i, pl.ds(j * L, L)] = cukey_f[pl.ds(t * L, L)]
        ceid_v[i, pl.ds(j * L, L)] = ceid_f[pl.ds(t * L, L)]
    _vloop(nst * 8, kbody)

    def drain(jj, carry):
        pltpu.sync_copy(ceid_v.at[jj], darr_hbm.at[cukey_v.at[jj]])
        return carry
    lax.fori_loop(0, nst, drain, 0)


def _k1b(r2, c2, eids2, bloom):
    return pl.kernel(
        _k1b_body,
        out_type=[
            jax.ShapeDtypeStruct((NN + E,), jnp.int32),
            jax.ShapeDtypeStruct((E // 128, 128), jnp.int32),
        ],
        mesh=_mesh,
        compiler_params=_sc_params,
        scratch_types=[
            pltpu.VMEM((CHR, 128), jnp.int32),
            pltpu.VMEM((CHR, 128), jnp.int32),
            pltpu.VMEM((CHR, 128), jnp.int32),
            pltpu.VMEM((CHR, 128), jnp.int32),
            pltpu.VMEM((CHR, 128), jnp.int32),
            pltpu.VMEM((CHR, 128), jnp.int32),
            pltpu.VMEM((CHR, 128), jnp.int32),
            pltpu.VMEM((CHR, 128), jnp.int32),
            pltpu.VMEM((CHR, 128), jnp.int32),
            pltpu.VMEM((CH + L,), jnp.int32),
            pltpu.VMEM((CH + L,), jnp.int32),
            pltpu.VMEM((CHR, 128), jnp.int32),
            pltpu.VMEM((CHR, 128), jnp.int32),
            pltpu.SemaphoreType.DMA,
        ],
    )(r2, c2, eids2, bloom)


def _k2_body(r_hbm, c_hbm, p_hbm, q_hbm, darr_hbm, eids_hbm, amb_hbm,
             w_hbm, spart_hbm,
             r_v, c_v, p_v, q_v, ukey_v, val_v, w_v, eid_v, amb_v,
             zero_v, s_sh, sem):
    cid, sid, wid = _wid()
    base = wid * CHR

    def zbody(i):
        zero_v[pl.ds(i * L, L)] = jnp.zeros((L,), jnp.float32)
    _vloop(4096 // L, zbody)
    pltpu.sync_copy(zero_v, s_sh.at[pl.ds(sid * 4096, 4096)])

    pltpu.sync_copy(r_hbm.at[pl.ds(base, CHR), :], r_v)
    pltpu.sync_copy(c_hbm.at[pl.ds(base, CHR), :], c_v)

    def ubody(t):
        i = t // 8
        j = lax.rem(t, 8)
        rr = r_v[i, pl.ds(j * L, L)]
        cc = c_v[i, pl.ds(j * L, L)]
        ukey_v[i, pl.ds(j * L, L)] = jnp.minimum(rr, cc) * N + jnp.maximum(rr, cc)
    _vloop(CH // L, ubody)

    descs = [pltpu.async_copy(darr_hbm.at[ukey_v.at[j]], w_v.at[j], sem)
             for j in range(CHR)]

    pltpu.sync_copy(p_hbm, p_v)
    pltpu.sync_copy(q_hbm, q_v)
    pltpu.sync_copy(eids_hbm.at[pl.ds(base, CHR), :], eid_v)
    pltpu.sync_copy(amb_hbm.at[pl.ds(base, CHR), :], amb_v)

    def vbody(t):
        i = t // 8
        j = lax.rem(t, 8)
        rr = r_v[i, pl.ds(j * L, L)]
        cc = c_v[i, pl.ds(j * L, L)]
        pv = plsc.load_gather(p_v, [rr])
        qv = plsc.load_gather(q_v, [cc])
        val_v[i, pl.ds(j * L, L)] = 1.0 / (1.0 + jnp.exp(-(pv + qv)))
    _vloop(CH // L, vbody)

    for d in descs:
        d.wait()

    def wbody(t):
        i = t // 8
        j = lax.rem(t, 8)
        sl = pl.ds(j * L, L)
        w_v[i, sl] = jnp.where(amb_v[i, sl] != 0, w_v[i, sl], eid_v[i, sl])
    _vloop(CH // L, wbody)

    pltpu.sync_copy(w_v, w_hbm.at[pl.ds(base, CHR), :])
    plsc.subcore_barrier()
    for j in range(CHR):
        pltpu.sync_copy(val_v.at[j], s_sh.at[w_v.at[j]], add=True)
    plsc.subcore_barrier()
    pltpu.sync_copy(s_sh.at[pl.ds(sid * 4096, 4096)],
                    spart_hbm.at[pl.ds(cid * E + sid * 4096, 4096)])


def _k2(r2, c2, p, q, darr, eids2, amb2):
    return pl.kernel(
        _k2_body,
        out_type=[
            jax.ShapeDtypeStruct((E // 128, 128), jnp.int32),
            jax.ShapeDtypeStruct((NC * E,), jnp.float32),
        ],
        mesh=_mesh,
        compiler_params=_sc_params,
        scratch_types=[
            pltpu.VMEM((CHR, 128), jnp.int32),
            pltpu.VMEM((CHR, 128), jnp.int32),
            pltpu.VMEM((N,), jnp.float32),
            pltpu.VMEM((N,), jnp.float32),
            pltpu.VMEM((CHR, 128), jnp.int32),
            pltpu.VMEM((CHR, 128), jnp.float32),
            pltpu.VMEM((CHR, 128), jnp.int32),
            pltpu.VMEM((CHR, 128), jnp.int32),
            pltpu.VMEM((CHR, 128), jnp.int32),
            pltpu.VMEM((4096,), jnp.float32),
            pltpu.VMEM_SHARED((E,), jnp.float32),
            pltpu.SemaphoreType.DMA,
        ],
    )(r2, c2, p, q, darr, eids2, amb2)


def _k4_body(r_hbm, c_hbm, d_hbm, w_hbm, spart_hbm, nid_hbm, z_hbm,
             outp_hbm, w0p_hbm,
             r_v, c_v, d_v, w_v, w2_v, s0_v, s1_v, coef_f, w0v_f,
             rows_a, rows_b, rows_c, zrows_v, w0z_v, nid_v,
             out_sh, w0_sh, sem, sem2, sem3):
    cid, sid, wid = _wid()
    base = wid * CHR

    def zbody(t):
        i = t // 8
        j = lax.rem(t, 8)
        zrows_v[i, pl.ds(j * L, L)] = jnp.zeros((L,), jnp.float32)
    _vloop(32 * D // L, zbody)

    def z2body(i):
        w0z_v[pl.ds(i * L, L)] = jnp.zeros((L,), jnp.float32)
    _vloop(256 // L, z2body)

    for j in range(8):
        pltpu.sync_copy(zrows_v, out_sh.at[pl.ds(sid * 256 + j * 32, 32), :])
    pltpu.sync_copy(w0z_v, w0_sh.at[pl.ds(sid * 256, 256)])

    pltpu.sync_copy(w_hbm.at[pl.ds(base, CHR), :], w_v)

    def abody(t):
        i = t // 8
        j = lax.rem(t, 8)
        w2_v[i, pl.ds(j * L, L)] = w_v[i, pl.ds(j * L, L)] + E
    _vloop(CH // L, abody)

    descs = ([pltpu.async_copy(spart_hbm.at[w_v.at[j]], s0_v.at[j], sem)
              for j in range(CHR)]
             + [pltpu.async_copy(spart_hbm.at[w2_v.at[j]], s1_v.at[j], sem)
                for j in range(CHR)])

    pltpu.sync_copy(r_hbm.at[pl.ds(base, CHR), :], r_v)
    pltpu.sync_copy(c_hbm.at[pl.ds(base, CHR), :], c_v)
    pltpu.sync_copy(d_hbm.at[pl.ds(base, CHR), :], d_v)
    pltpu.sync_copy(nid_hbm, nid_v)

    def _gather(sub, buf):
        return [pltpu.async_copy(z_hbm.at[c_v.at[sub * SUBR + j]],
                                 buf.at[pl.ds(j * 128, 128), :], sem2)
                for j in range(SUBR)]

    gd = {}
    gd[0] = _gather(0, rows_a)
    gd[1] = _gather(1, rows_b)

    for d in descs:
        d.wait()

    def cbody(t):
        i = t // 8
        j = lax.rem(t, 8)
        sl = pl.ds(j * L, L)
        rr = r_v[i, sl]
        cc = c_v[i, sl]
        s = s0_v[i, sl] + s1_v[i, sl]
        co = 0.5 * s * d_v[i, sl]
        co = jnp.where(rr == cc, 0.0, co)
        coef_f[pl.ds(t * L, L)] = co
        w0v_f[pl.ds(t * L, L)] = jnp.where(rr == nid_v[...], co, 0.0)
    _vloop(CH // L, cbody)

    plsc.subcore_barrier()

    def _scale(sub, buf):
        def sbody(h):
            i = h * 4
            cbs = [plsc.load_gather(
                coef_f, [jnp.zeros((L,), jnp.int32) + (sub * SUB + i + k)])
                for k in range(4)]
            for k in range(4):
                for jj in range(D // L):
                    sl = pl.ds(jj * L, L)
                    buf[i + k, sl] = buf[i + k, sl] * cbs[k]
        _vloop(SUB // 4, sbody)

    def _scatter(sub, buf):
        return [pltpu.async_copy(buf.at[pl.ds(j * 128, 128), :],
                                 out_sh.at[r_v.at[sub * SUBR + j]], sem3,
                                 add=True)
                for j in range(SUBR)]

    bufs = [rows_a, rows_b, rows_c]
    sd = {}
    for sub in range(NSUB):
        for d in gd[sub]:
            d.wait()
        _scale(sub, bufs[sub % 3])
        sd[sub] = _scatter(sub, bufs[sub % 3])
        if sub + 2 < NSUB:
            if sub - 1 >= 0:
                for d in sd[sub - 1]:
                    d.wait()
            gd[sub + 2] = _gather(sub + 2, bufs[(sub + 2) % 3])
    for sub in range(max(0, NSUB - 3), NSUB):
        for d in sd[sub]:
            d.wait()

    for j in range(CHR):
        pltpu.sync_copy(w0v_f.at[pl.ds(j * 128, 128)],
                        w0_sh.at[c_v.at[j]], add=True)
    plsc.subcore_barrier()

    pltpu.sync_copy(out_sh.at[pl.ds(sid * 256, 256), :],
                    outp_hbm.at[pl.ds(cid * N + sid * 256, 256), :])
    pltpu.sync_copy(w0_sh.at[pl.ds(sid * 256, 256)],
                    w0p_hbm.at[pl.ds(cid * N + sid * 256, 256)])


def _k4(r2, c2, data2, w2d, spart, nid, z):
    return pl.kernel(
        _k4_body,
        out_type=[
            jax.ShapeDtypeStruct((NC * N, D), jnp.float32),
            jax.ShapeDtypeStruct((NC * N,), jnp.float32),
        ],
        mesh=_mesh,
        compiler_params=_sc_params,
        scratch_types=[
            pltpu.VMEM((CHR, 128), jnp.int32),
            pltpu.VMEM((CHR, 128), jnp.int32),
            pltpu.VMEM((CHR, 128), jnp.float32),
            pltpu.VMEM((CHR, 128), jnp.int32),
            pltpu.VMEM((CHR, 128), jnp.int32),
            pltpu.VMEM((CHR, 128), jnp.float32),
            pltpu.VMEM((CHR, 128), jnp.float32),
            pltpu.VMEM((CH,), jnp.float32),
            pltpu.VMEM((CH,), jnp.float32),
            pltpu.VMEM((SUB, D), jnp.float32),
            pltpu.VMEM((SUB, D), jnp.float32),
            pltpu.VMEM((SUB, D), jnp.float32),
            pltpu.VMEM((32, D), jnp.float32),
            pltpu.VMEM((256,), jnp.float32),
            pltpu.VMEM((L,), jnp.int32),
            pltpu.VMEM_SHARED((N, D), jnp.float32),
            pltpu.VMEM_SHARED((N,), jnp.float32),
            pltpu.SemaphoreType.DMA,
            pltpu.SemaphoreType.DMA,
            pltpu.SemaphoreType.DMA,
        ],
    )(r2, c2, data2, w2d, spart, nid, z)


def _k5_body(out0_ref, out1_ref, w0a_ref, w0b_ref, w2_ref, res_ref, acc_ref):
    i = pl.program_id(0)
    h = jnp.maximum(out0_ref[...] + out1_ref[...], 0.0)
    wv = w0a_ref[...] + w0b_ref[...]
    contrib = jnp.sum(wv * h, axis=0, keepdims=True)

    @pl.when(i == 0)
    def _():
        acc_ref[...] = contrib

    @pl.when(i != 0)
    def _():
        acc_ref[...] = acc_ref[...] + contrib

    @pl.when(i == NBLK - 1)
    def _():
        r16 = lax.dot_general(
            acc_ref[...], w2_ref[...], (((1,), (0,)), ((), ())),
            preferred_element_type=jnp.float32)
        m = jnp.max(r16)
        e = jnp.exp(r16 - m)
        res_ref[...] = e / jnp.sum(e)


def _k5(outp, w0p2, W2):
    return pl.pallas_call(
        _k5_body,
        grid=(NBLK,),
        in_specs=[
            pl.BlockSpec((ROWBLK, D), lambda i: (i, 0)),
            pl.BlockSpec((ROWBLK, D), lambda i: (i + NBLK, 0)),
            pl.BlockSpec((ROWBLK, 1), lambda i: (i, 0)),
            pl.BlockSpec((ROWBLK, 1), lambda i: (i + NBLK, 0)),
            pl.BlockSpec((D, C), lambda i: (0, 0)),
        ],
        out_specs=pl.BlockSpec((1, C), lambda i: (0, 0)),
        out_shape=jax.ShapeDtypeStruct((1, C), jnp.float32),
        scratch_shapes=[pltpu.VMEM((1, D), jnp.float32)],
    )(outp, outp, w0p2, w0p2, W2)


def kernel(x, embed, adj_row, adj_col, adj_data, nodeid, sub_new_edge_index,
           tmp, W_mask, b_mask, W1, W2):
    nodeid = jnp.asarray(nodeid, jnp.int32)
    c0 = (embed[nodeid] @ W_mask[2 * D:, 0] + b_mask[0]).reshape(1, 1)
    p2, q2, z = _k0(c0, embed, x, W_mask, W1)
    p, q = p2.reshape(N), q2.reshape(N)

    r2 = adj_row.astype(jnp.int32).reshape(E // 128, 128)
    c2 = adj_col.astype(jnp.int32).reshape(E // 128, 128)
    data2 = adj_data.astype(jnp.float32).reshape(E // 128, 128)
    nid = jnp.broadcast_to(nodeid, (L,)).astype(jnp.int32)

    eids2 = jnp.arange(E, dtype=jnp.int32).reshape(E // 128, 128)
    bloom = _k1a(r2, c2)
    darr, amb2 = _k1b(r2, c2, eids2, bloom)
    w2d, spart = _k2(r2, c2, p, q, darr, eids2, amb2)
    outp, w0p = _k4(r2, c2, data2, w2d, spart, nid, z)

    res = _k5(outp, w0p.reshape(NC * N, 1), W2)
    return res.reshape(C)
